# baseline (speedup 1.0000x reference)
import functools
import math

import jax
import jax.numpy as jnp
from jax.experimental import pallas as pl

N = 20
IN_DIM = 9
HID = 128
K = 5
EPS = 1e-5
BIG = 3.0e38


def _ln(x, g, b):
    mu = jnp.mean(x, axis=-1, keepdims=True)
    ms = jnp.mean(x * x, axis=-1, keepdims=True)
    var = ms - mu * mu
    return (x - mu) * jax.lax.rsqrt(var + EPS) * g + b


def _gelu(x):
    return 0.5 * x * (1.0 + jax.lax.erf(x * (1.0 / math.sqrt(2.0))))


def _kernel(x_ref, we_ref, be_ref, w1_ref, b1_ref, w2_ref, b2_ref,
            g1_ref, be1_ref, g2_ref, be2_ref, out_ref):
    x = x_ref[...]
    pos = x[:, :, 6:9]

    n = jnp.sum(pos * pos, axis=-1, keepdims=True)
    ones = jnp.ones_like(n)
    lhs = jnp.concatenate([-2.0 * pos, ones], axis=-1)
    rhs = jnp.concatenate([pos, n], axis=-1)
    d2 = jax.lax.dot_general(
        lhs, rhs, (((2,), (2,)), ((0,), (0,))),
        precision=jax.lax.Precision.HIGHEST,
        preferred_element_type=jnp.float32)

    work = d2
    for _ in range(K - 1):
        m = jnp.min(work, axis=-1, keepdims=True)
        work = jnp.where(work <= m, BIG, work)
    thr = jnp.min(work, axis=-1, keepdims=True)
    adj = (d2 <= thr).astype(jnp.bfloat16)

    h = jax.lax.dot_general(
        x.astype(jnp.bfloat16), we_ref[...].astype(jnp.bfloat16),
        (((2,), (0,)), ((), ())),
        preferred_element_type=jnp.float32) + be_ref[...]

    def agg(a, hh):
        return jax.lax.dot_general(
            a, hh.astype(jnp.bfloat16), (((2,), (1,)), ((0,), (0,))),
            preferred_element_type=jnp.float32)

    h = agg(adj, h)
    h = jax.lax.dot_general(
        h.astype(jnp.bfloat16), w1_ref[...].astype(jnp.bfloat16),
        (((2,), (0,)), ((), ())),
        preferred_element_type=jnp.float32) + b1_ref[...]
    h = _gelu(_ln(h, g1_ref[...], be1_ref[...]))

    h2 = agg(adj, h)
    h2 = jax.lax.dot_general(
        h2.astype(jnp.bfloat16), w2_ref[...].astype(jnp.bfloat16),
        (((2,), (0,)), ((), ())),
        preferred_element_type=jnp.float32) + b2_ref[...]
    h2 = _ln(h2, g2_ref[...], be2_ref[...])
    h = _gelu(h + h2)

    out_ref[...] = jnp.max(h, axis=1)


@jax.jit
def kernel(hbond_coords, W_embed, b_embed, W1, b1, W2, b2, g1, beta1, g2, beta2):
    B = hbond_coords.shape[0]
    G = 128
    grid = (B // G,)

    def blk(i):
        return (i, 0, 0)

    def const2(i):
        return (0, 0)

    out = pl.pallas_call(
        _kernel,
        grid=grid,
        in_specs=[
            pl.BlockSpec((G, N, IN_DIM), blk),
            pl.BlockSpec((IN_DIM, HID), const2),
            pl.BlockSpec((1, HID), const2),
            pl.BlockSpec((HID, HID), const2),
            pl.BlockSpec((1, HID), const2),
            pl.BlockSpec((HID, HID), const2),
            pl.BlockSpec((1, HID), const2),
            pl.BlockSpec((1, HID), const2),
            pl.BlockSpec((1, HID), const2),
            pl.BlockSpec((1, HID), const2),
            pl.BlockSpec((1, HID), const2),
        ],
        out_specs=pl.BlockSpec((G, HID), lambda i: (i, 0)),
        out_shape=jax.ShapeDtypeStruct((B, HID), jnp.float32),
    )(hbond_coords.reshape(B, N, IN_DIM), W_embed,
      b_embed.reshape(1, HID), W1, b1.reshape(1, HID), W2,
      b2.reshape(1, HID), g1.reshape(1, HID), beta1.reshape(1, HID),
      g2.reshape(1, HID), beta2.reshape(1, HID))
    return out


# speedup vs baseline: 1.8591x; 1.8591x over previous
import math

import jax
import jax.numpy as jnp
from jax.experimental import pallas as pl

N = 20
IN_DIM = 9
HID = 128
K = 5
EPS = 1e-5
BIG = 3.0e38


def _ln(x, g, b):
    mu = jnp.mean(x, axis=-1, keepdims=True)
    ms = jnp.mean(x * x, axis=-1, keepdims=True)
    var = ms - mu * mu
    return (x - mu) * jax.lax.rsqrt(var + EPS) * g + b


def _gelu(x):
    return 0.5 * x * (1.0 + jax.lax.erf(x * (1.0 / math.sqrt(2.0))))


def _kernel(x_ref, pt_ref, we_ref, be_ref, w1_ref, b1_ref, w2_ref, b2_ref,
            g1_ref, be1_ref, g2_ref, be2_ref, out_ref):
    pt = pt_ref[...]

    d2 = jnp.zeros((N, N, pt.shape[2]), jnp.float32)
    for c in range(3):
        pc = pt[c]
        diff = pc[:, None, :] - pc[None, :, :]
        d2 = d2 + diff * diff

    work = d2
    for _ in range(K - 1):
        m = jnp.min(work, axis=1, keepdims=True)
        work = jnp.where(work <= m, BIG, work)
    thr = jnp.min(work, axis=1, keepdims=True)
    adj = (d2 <= thr).astype(jnp.float32)

    x = x_ref[...]
    h = jax.lax.dot_general(
        x, we_ref[...], (((2,), (0,)), ((), ())),
        preferred_element_type=jnp.float32) + be_ref[...]

    def agg(a, hh):
        return jax.lax.dot_general(
            a, hh, (((1,), (1,)), ((2,), (0,))),
            preferred_element_type=jnp.float32)

    h = agg(adj, h)
    h = jax.lax.dot_general(
        h, w1_ref[...], (((2,), (0,)), ((), ())),
        preferred_element_type=jnp.float32) + b1_ref[...]
    h = _gelu(_ln(h, g1_ref[...], be1_ref[...]))

    h2 = agg(adj, h)
    h2 = jax.lax.dot_general(
        h2, w2_ref[...], (((2,), (0,)), ((), ())),
        preferred_element_type=jnp.float32) + b2_ref[...]
    h2 = _ln(h2, g2_ref[...], be2_ref[...])
    h = _gelu(h + h2)

    out_ref[...] = jnp.max(h, axis=1)


@jax.jit
def kernel(hbond_coords, W_embed, b_embed, W1, b1, W2, b2, g1, beta1, g2, beta2):
    B = hbond_coords.shape[0]
    G = 128
    grid = (B // G,)

    pos_t = jnp.transpose(hbond_coords[:, :, 6:9], (2, 1, 0))

    def const2(i):
        return (0, 0)

    out = pl.pallas_call(
        _kernel,
        grid=grid,
        in_specs=[
            pl.BlockSpec((G, N, IN_DIM), lambda i: (i, 0, 0)),
            pl.BlockSpec((3, N, G), lambda i: (0, 0, i)),
            pl.BlockSpec((IN_DIM, HID), const2),
            pl.BlockSpec((1, HID), const2),
            pl.BlockSpec((HID, HID), const2),
            pl.BlockSpec((1, HID), const2),
            pl.BlockSpec((HID, HID), const2),
            pl.BlockSpec((1, HID), const2),
            pl.BlockSpec((1, HID), const2),
            pl.BlockSpec((1, HID), const2),
            pl.BlockSpec((1, HID), const2),
            pl.BlockSpec((1, HID), const2),
        ],
        out_specs=pl.BlockSpec((G, HID), lambda i: (i, 0)),
        out_shape=jax.ShapeDtypeStruct((B, HID), jnp.float32),
    )(hbond_coords.reshape(B, N, IN_DIM), pos_t, W_embed,
      b_embed.reshape(1, HID), W1, b1.reshape(1, HID), W2,
      b2.reshape(1, HID), g1.reshape(1, HID), beta1.reshape(1, HID),
      g2.reshape(1, HID), beta2.reshape(1, HID))
    return out


# speedup vs baseline: 2.2084x; 1.1878x over previous
import math

import jax
import jax.numpy as jnp
from jax.experimental import pallas as pl

N = 20
IN_DIM = 9
HID = 128
K = 5
EPS = 1e-5
BIG = 3.0e38


def _ln(x, g, b):
    mu = jnp.mean(x, axis=-1, keepdims=True, dtype=jnp.float32)
    ms = jnp.mean(x.astype(jnp.float32) * x, axis=-1, keepdims=True,
                  dtype=jnp.float32)
    var = ms - mu * mu
    s = jax.lax.rsqrt(var + EPS)
    return (x - mu.astype(jnp.bfloat16)) * (s.astype(jnp.bfloat16) * g) + b


def _gelu(x):
    return 0.5 * x * (1.0 + jax.lax.erf(x * jnp.bfloat16(1.0 / math.sqrt(2.0))))


def _kernel(x_ref, pt_ref, we_ref, be_ref, w1_ref, b1_ref, w2_ref, b2_ref,
            g1_ref, be1_ref, g2_ref, be2_ref, out_ref):
    pt = pt_ref[...]

    d2 = jnp.zeros((N, N, pt.shape[2]), jnp.float32)
    for c in range(3):
        pc = pt[c]
        diff = pc[:, None, :] - pc[None, :, :]
        d2 = d2 + diff * diff

    work = d2
    for _ in range(K - 1):
        m = jnp.min(work, axis=1, keepdims=True)
        work = jnp.where(work <= m, BIG, work)
    thr = jnp.min(work, axis=1, keepdims=True)
    adj = (d2 <= thr).astype(jnp.bfloat16)

    x = x_ref[...]
    h = (jax.lax.dot_general(
        x, we_ref[...], (((2,), (0,)), ((), ())),
        preferred_element_type=jnp.float32) + be_ref[...].astype(jnp.float32)
         ).astype(jnp.bfloat16)

    def agg(a, hh):
        return jax.lax.dot_general(
            a, hh, (((1,), (1,)), ((2,), (0,))),
            preferred_element_type=jnp.float32).astype(jnp.bfloat16)

    h = agg(adj, h)
    h = (jax.lax.dot_general(
        h, w1_ref[...], (((2,), (0,)), ((), ())),
        preferred_element_type=jnp.float32) + b1_ref[...].astype(jnp.float32)
         ).astype(jnp.bfloat16)
    h = _gelu(_ln(h, g1_ref[...], be1_ref[...]))

    h2 = agg(adj, h)
    h2 = (jax.lax.dot_general(
        h2, w2_ref[...], (((2,), (0,)), ((), ())),
        preferred_element_type=jnp.float32) + b2_ref[...].astype(jnp.float32)
          ).astype(jnp.bfloat16)
    h2 = _ln(h2, g2_ref[...], be2_ref[...])
    h = _gelu(h + h2)

    out_ref[...] = jnp.max(h, axis=1).astype(jnp.float32)


@jax.jit
def kernel(hbond_coords, W_embed, b_embed, W1, b1, W2, b2, g1, beta1, g2, beta2):
    B = hbond_coords.shape[0]
    G = 128
    grid = (B // G,)

    pos_t = jnp.transpose(hbond_coords[:, :, 6:9], (2, 1, 0))
    bf = jnp.bfloat16

    def const2(i):
        return (0, 0)

    out = pl.pallas_call(
        _kernel,
        grid=grid,
        in_specs=[
            pl.BlockSpec((G, N, IN_DIM), lambda i: (i, 0, 0)),
            pl.BlockSpec((3, N, G), lambda i: (0, 0, i)),
            pl.BlockSpec((IN_DIM, HID), const2),
            pl.BlockSpec((1, HID), const2),
            pl.BlockSpec((HID, HID), const2),
            pl.BlockSpec((1, HID), const2),
            pl.BlockSpec((HID, HID), const2),
            pl.BlockSpec((1, HID), const2),
            pl.BlockSpec((1, HID), const2),
            pl.BlockSpec((1, HID), const2),
            pl.BlockSpec((1, HID), const2),
            pl.BlockSpec((1, HID), const2),
        ],
        out_specs=pl.BlockSpec((G, HID), lambda i: (i, 0)),
        out_shape=jax.ShapeDtypeStruct((B, HID), jnp.float32),
    )(hbond_coords.reshape(B, N, IN_DIM).astype(bf), pos_t,
      W_embed.astype(bf), b_embed.reshape(1, HID).astype(bf),
      W1.astype(bf), b1.reshape(1, HID).astype(bf),
      W2.astype(bf), b2.reshape(1, HID).astype(bf),
      g1.reshape(1, HID).astype(bf), beta1.reshape(1, HID).astype(bf),
      g2.reshape(1, HID).astype(bf), beta2.reshape(1, HID).astype(bf))
    return out
